# baseline (speedup 1.0000x reference)
import functools

import jax
import jax.numpy as jnp
from jax import lax
from jax.experimental import pallas as pl
from jax.experimental.pallas import tpu as pltpu
from jax.experimental.pallas import tpu_sc as plsc

D = 32
HIST = 50
CB = 128
CPH = 4
NBUF = 8
NW = 32


@functools.partial(jax.jit, static_argnames=("batch",))
def _sc_gather_t(idx, table, batch):
    rows_per_w = batch // NW
    nunits = HIST * CPH
    ngroups = nunits // NBUF
    mesh = plsc.VectorSubcoreMesh(core_axis_name="c", subcore_axis_name="s")

    @functools.partial(
        pl.kernel,
        mesh=mesh,
        out_type=jax.ShapeDtypeStruct((HIST, batch, D), jnp.float32),
        scratch_types=[
            pltpu.VMEM((rows_per_w, HIST), jnp.int32),
            pltpu.VMEM((HIST * rows_per_w,), jnp.int32),
            pltpu.VMEM((NBUF, CB, D), jnp.float32),
            [pltpu.SemaphoreType.DMA] * NBUF,
            [pltpu.SemaphoreType.DMA] * NBUF,
        ],
        compiler_params=pltpu.CompilerParams(
            use_tc_tiling_on_sc=False, needs_layout_passes=False
        ),
    )
    def k(idx_hbm, table_hbm, out_hbm, idx_v, idx_t, gbuf, gsems, wsems):
        wid = lax.axis_index("s") * 2 + lax.axis_index("c")
        base = wid * rows_per_w
        iota = lax.iota(jnp.int32, 16)

        pltpu.sync_copy(idx_hbm.at[pl.ds(base, rows_per_w)], idx_v)

        def trow(r, carry):
            for o in (0, 16, 32, 34):
                vals = idx_v[r, pl.ds(o, 16)]
                dst = (o + iota) * rows_per_w + r
                plsc.store_scatter(idx_t, [dst], vals)
            return carry

        lax.fori_loop(0, rows_per_w, trow, 0)

        def start_gather(u, s):
            pltpu.async_copy(
                table_hbm.at[idx_t.at[pl.ds(u * CB, CB)]], gbuf.at[s], gsems[s]
            )

        def wait_gather(s):
            pltpu.make_async_copy(
                table_hbm.at[pl.ds(0, CB)], gbuf.at[s], gsems[s]
            ).wait()

        def start_write(h, c, s):
            pltpu.async_copy(
                gbuf.at[s],
                out_hbm.at[h, pl.ds(base + c * CB, CB), :],
                wsems[s],
            )

        def wait_write(s):
            pltpu.make_async_copy(
                gbuf.at[s], out_hbm.at[0, pl.ds(base, CB), :], wsems[s]
            ).wait()

        for s in range(NBUF):
            start_gather(s, s)

        def body(g, carry):
            for s in range(NBUF):
                h = 2 * g + s // CPH
                c = s % CPH
                wait_gather(s)
                start_write(h, c, s)

                @pl.when(g < ngroups - 1)
                def _():
                    wait_write(s)
                    start_gather(NBUF * (g + 1) + s, s)

            return carry

        lax.fori_loop(0, ngroups, body, 0)
        for s in range(NBUF):
            wait_write(s)

    return k(idx, table)


def kernel(inputs, embeddings):
    batch, _ = inputs.shape
    out_t = _sc_gather_t(inputs.astype(jnp.int32), embeddings, batch)
    return jnp.transpose(out_t, (1, 0, 2))
